# Initial kernel scaffold; baseline (speedup 1.0000x reference)
import functools

import jax
import jax.numpy as jnp
from jax import lax
from jax.experimental import pallas as pl
from jax.experimental.pallas import tpu as pltpu
from jax.experimental.pallas import tpu_sc as plsc

_B = 8
_N = 10000
_E = 160000
_D = 16
_SL = 128
_STL = 16

_NC = 2
_NS = 16
_BPC = _B // _NC
_CHUNK = _E // _NS
_NSUB = 5
_SUB = _CHUNK // _NSUB

_WR_TILES = 10
_WR_ROWS = _N // _WR_TILES


def _sc_scatter_body(bonds_h, idx_h, zer_h, one_h, sums_h, cnt_h,
                     idx_v, row_v, ones_v, acc_s, cnt_s):
    c = lax.axis_index("c")
    s = lax.axis_index("s")
    base_e = s * _CHUNK

    pltpu.sync_copy(one_h, ones_v)
    for j in range(_NSUB):
        pltpu.sync_copy(idx_h.at[pl.ds(base_e + j * _SUB, _SUB)], idx_v.at[j])

    @pl.when(s < _WR_TILES)
    def _zero():
        r0 = s * _WR_ROWS
        for bl in range(_BPC):
            pltpu.sync_copy(zer_h, acc_s.at[bl, pl.ds(r0, _WR_ROWS)])
        pltpu.sync_copy(zer_h, cnt_s.at[pl.ds(r0, _WR_ROWS)])

    plsc.subcore_barrier()

    for j in range(_NSUB):
        pltpu.sync_copy(ones_v, cnt_s.at[idx_v.at[j]], add=True)

    for bl in range(_BPC):
        b = c * _BPC + bl
        for j in range(_NSUB):
            pltpu.sync_copy(bonds_h.at[b, pl.ds(base_e + j * _SUB, _SUB)],
                            row_v)
            pltpu.sync_copy(row_v, acc_s.at[bl].at[idx_v.at[j]], add=True)

    plsc.subcore_barrier()

    @pl.when(s < _WR_TILES)
    def _write():
        r0 = s * _WR_ROWS
        for bl in range(_BPC):
            b = c * _BPC + bl
            pltpu.sync_copy(acc_s.at[bl, pl.ds(r0, _WR_ROWS)],
                            sums_h.at[b, pl.ds(r0, _WR_ROWS)])

        @pl.when(c == 0)
        def _():
            pltpu.sync_copy(cnt_s.at[pl.ds(r0, _WR_ROWS)],
                            cnt_h.at[pl.ds(r0, _WR_ROWS)])


_sc_scatter = functools.partial(
    pl.kernel,
    out_type=[
        jax.ShapeDtypeStruct((_B, _N, _D), jnp.float32),
        jax.ShapeDtypeStruct((_N, _D), jnp.float32),
    ],
    mesh=plsc.VectorSubcoreMesh(core_axis_name="c", subcore_axis_name="s",
                                num_cores=_NC, num_subcores=_NS),
    scratch_types=[
        pltpu.VMEM((_NSUB, _SUB), jnp.int32),
        pltpu.VMEM((_SUB, _D), jnp.float32),
        pltpu.VMEM((_SUB, _D), jnp.float32),
        pltpu.VMEM_SHARED((_BPC, _N, _D), jnp.float32),
        pltpu.VMEM_SHARED((_N, _D), jnp.float32),
    ],
)(_sc_scatter_body)


_BLK = 2500


def _mlp_body(sums_r, cnt_r, sites_r, states_r,
              w1_r, b1_r, w2_r, b2_r, w3_r, b3_r, out_r):
    b = pl.program_id(0)
    inv = 1.0 / jnp.maximum(cnt_r[:, 0:1], 1.0)
    pool = sums_r[0] * inv

    sel = (lax.broadcasted_iota(jnp.int32, (1, _B), 1) == b)
    st_all = jnp.dot(states_r[...], w1_r[_D + _SL:, :],
                     preferred_element_type=jnp.float32)
    x_state = jnp.dot(sel.astype(jnp.float32), st_all,
                      preferred_element_type=jnp.float32)

    h = (jnp.dot(pool, w1_r[0:_D, :], preferred_element_type=jnp.float32)
         + jnp.dot(sites_r[0], w1_r[_D:_D + _SL, :],
                   preferred_element_type=jnp.float32)
         + x_state + b1_r[...])
    h = jnp.maximum(h, 0.0)
    h = jnp.maximum(jnp.dot(h, w2_r[...], preferred_element_type=jnp.float32)
                    + b2_r[...], 0.0)
    h = jnp.maximum(jnp.dot(h, w3_r[...], preferred_element_type=jnp.float32)
                    + b3_r[...], 0.0)
    out_r[0] = h


def _mlp_call(sums, cnt, sites, states, w1, b1, w2, b2, w3, b3):
    d_in = _D + _SL + _STL
    grid = (_B, _N // _BLK)
    return pl.pallas_call(
        _mlp_body,
        grid=grid,
        in_specs=[
            pl.BlockSpec((1, _BLK, _D), lambda b, n: (b, n, 0)),
            pl.BlockSpec((_BLK, _D), lambda b, n: (n, 0)),
            pl.BlockSpec((1, _BLK, _SL), lambda b, n: (b, n, 0)),
            pl.BlockSpec((_B, _STL), lambda b, n: (0, 0)),
            pl.BlockSpec((d_in, 128), lambda b, n: (0, 0)),
            pl.BlockSpec((1, 128), lambda b, n: (0, 0)),
            pl.BlockSpec((128, 128), lambda b, n: (0, 0)),
            pl.BlockSpec((1, 128), lambda b, n: (0, 0)),
            pl.BlockSpec((128, _SL), lambda b, n: (0, 0)),
            pl.BlockSpec((1, _SL), lambda b, n: (0, 0)),
        ],
        out_specs=pl.BlockSpec((1, _BLK, _SL), lambda b, n: (b, n, 0)),
        out_shape=jax.ShapeDtypeStruct((_B, _N, _SL), jnp.float32),
    )(sums, cnt, sites, states, w1, b1, w2, b2, w3, b3)


def kernel(sites, bonds, states, indices1, W1, b1, W2, b2, W3, b3):
    zeros = jnp.zeros((_WR_ROWS, _D), jnp.float32)
    ones = jnp.ones((_SUB, _D), jnp.float32)
    sums, cnt = _sc_scatter(bonds, indices1, zeros, ones)
    return _mlp_call(sums, cnt, sites, states,
                     W1, b1.reshape(1, -1), W2, b2.reshape(1, -1),
                     W3, b3.reshape(1, -1))


# speedup vs baseline: 25.8635x; 25.8635x over previous
import functools

import jax
import jax.numpy as jnp
from jax import lax
from jax.experimental import pallas as pl
from jax.experimental.pallas import tpu as pltpu
from jax.experimental.pallas import tpu_sc as plsc

_B = 8
_N = 10000
_E = 160000
_D = 16
_SL = 128
_STL = 16

_NC = 2
_NS = 16
_BPC = _B // _NC
_CHUNK = _E // _NS
_NSUB = 5
_SUB = _CHUNK // _NSUB

_WR_TILES = 10
_WR_ROWS = _N // _WR_TILES


def _sc_scatter_body(bonds_h, idx_h, zer_h, one_h, sums_h, cnt_h,
                     idx_v0, idx_v1, idx_v2, idx_v3, idx_v4,
                     row_v, ones_v, acc_s, cnt_s):
    idx_v = [idx_v0, idx_v1, idx_v2, idx_v3, idx_v4]
    c = lax.axis_index("c")
    s = lax.axis_index("s")
    base_e = s * _CHUNK

    pltpu.sync_copy(one_h, ones_v)
    for j in range(_NSUB):
        pltpu.sync_copy(idx_h.at[pl.ds(base_e + j * _SUB, _SUB)], idx_v[j])

    @pl.when(s < _WR_TILES)
    def _zero():
        r0 = s * _WR_ROWS
        for bl in range(_BPC):
            pltpu.sync_copy(zer_h, acc_s.at[bl, pl.ds(r0, _WR_ROWS)])
        pltpu.sync_copy(zer_h, cnt_s.at[pl.ds(r0, _WR_ROWS)])

    plsc.subcore_barrier()

    for j in range(_NSUB):
        pltpu.sync_copy(ones_v, cnt_s.at[idx_v[j]], add=True)

    for bl in range(_BPC):
        b = c * _BPC + bl
        for j in range(_NSUB):
            pltpu.sync_copy(bonds_h.at[b, pl.ds(base_e + j * _SUB, _SUB)],
                            row_v)
            pltpu.sync_copy(row_v, acc_s.at[bl].at[idx_v[j]], add=True)

    plsc.subcore_barrier()

    @pl.when(s < _WR_TILES)
    def _write():
        r0 = s * _WR_ROWS
        for bl in range(_BPC):
            b = c * _BPC + bl
            pltpu.sync_copy(acc_s.at[bl, pl.ds(r0, _WR_ROWS)],
                            sums_h.at[b, pl.ds(r0, _WR_ROWS)])

        @pl.when(c == 0)
        def _():
            pltpu.sync_copy(cnt_s.at[pl.ds(r0, _WR_ROWS)],
                            cnt_h.at[pl.ds(r0, _WR_ROWS)])


@functools.cache
def _sc_scatter():
    return pl.kernel(
        _sc_scatter_body,
        out_type=[
            jax.ShapeDtypeStruct((_B, _N, _D), jnp.float32),
            jax.ShapeDtypeStruct((_N, _D), jnp.float32),
        ],
        mesh=plsc.VectorSubcoreMesh(core_axis_name="c", subcore_axis_name="s",
                                    num_cores=_NC, num_subcores=_NS),
        compiler_params=pltpu.CompilerParams(use_tc_tiling_on_sc=False),
        scratch_types=[
            pltpu.VMEM((_SUB,), jnp.int32),
            pltpu.VMEM((_SUB,), jnp.int32),
            pltpu.VMEM((_SUB,), jnp.int32),
            pltpu.VMEM((_SUB,), jnp.int32),
            pltpu.VMEM((_SUB,), jnp.int32),
            pltpu.VMEM((_SUB, _D), jnp.float32),
            pltpu.VMEM((_SUB, _D), jnp.float32),
            pltpu.VMEM_SHARED((_BPC, _N, _D), jnp.float32),
            pltpu.VMEM_SHARED((_N, _D), jnp.float32),
        ],
    )


_BLK = 2000


def _mlp_body(sums_r, cnt_r, sites_r, states_r,
              w1_r, b1_r, w2_r, b2_r, w3_r, b3_r, out_r):
    b = pl.program_id(0)
    inv = 1.0 / jnp.maximum(cnt_r[:, 0:1], 1.0)
    pool = sums_r[0] * inv

    sel = (lax.broadcasted_iota(jnp.int32, (1, _B), 1) == b)
    st_all = jnp.dot(states_r[...], w1_r[_D + _SL:, :],
                     preferred_element_type=jnp.float32)
    x_state = jnp.dot(sel.astype(jnp.float32), st_all,
                      preferred_element_type=jnp.float32)

    h = (jnp.dot(pool, w1_r[0:_D, :], preferred_element_type=jnp.float32)
         + jnp.dot(sites_r[0], w1_r[_D:_D + _SL, :],
                   preferred_element_type=jnp.float32)
         + x_state + b1_r[...])
    h = jnp.maximum(h, 0.0)
    h = jnp.maximum(jnp.dot(h, w2_r[...], preferred_element_type=jnp.float32)
                    + b2_r[...], 0.0)
    h = jnp.maximum(jnp.dot(h, w3_r[...], preferred_element_type=jnp.float32)
                    + b3_r[...], 0.0)
    out_r[0] = h


def _mlp_call(sums, cnt, sites, states, w1, b1, w2, b2, w3, b3):
    d_in = _D + _SL + _STL
    grid = (_B, _N // _BLK)
    return pl.pallas_call(
        _mlp_body,
        grid=grid,
        in_specs=[
            pl.BlockSpec((1, _BLK, _D), lambda b, n: (b, n, 0)),
            pl.BlockSpec((_BLK, _D), lambda b, n: (n, 0)),
            pl.BlockSpec((1, _BLK, _SL), lambda b, n: (b, n, 0)),
            pl.BlockSpec((_B, _STL), lambda b, n: (0, 0)),
            pl.BlockSpec((d_in, 128), lambda b, n: (0, 0)),
            pl.BlockSpec((1, 128), lambda b, n: (0, 0)),
            pl.BlockSpec((128, 128), lambda b, n: (0, 0)),
            pl.BlockSpec((1, 128), lambda b, n: (0, 0)),
            pl.BlockSpec((128, _SL), lambda b, n: (0, 0)),
            pl.BlockSpec((1, _SL), lambda b, n: (0, 0)),
        ],
        out_specs=pl.BlockSpec((1, _BLK, _SL), lambda b, n: (b, n, 0)),
        out_shape=jax.ShapeDtypeStruct((_B, _N, _SL), jnp.float32),
    )(sums, cnt, sites, states, w1, b1, w2, b2, w3, b3)


def kernel(sites, bonds, states, indices1, W1, b1, W2, b2, W3, b3):
    zeros = jnp.zeros((_WR_ROWS, _D), jnp.float32)
    ones = jnp.ones((_SUB, _D), jnp.float32)
    sums, cnt = _sc_scatter()(bonds, indices1, zeros, ones)
    return _mlp_call(sums, cnt, sites, states,
                     W1, b1.reshape(1, -1), W2, b2.reshape(1, -1),
                     W3, b3.reshape(1, -1))


# speedup vs baseline: 47.0838x; 1.8205x over previous
import functools

import jax
import jax.numpy as jnp
from jax import lax
from jax.experimental import pallas as pl
from jax.experimental.pallas import tpu as pltpu
from jax.experimental.pallas import tpu_sc as plsc

_B = 8
_N = 10000
_E = 160000
_D = 16
_SL = 128
_STL = 16

_NC = 2
_NS = 16
_FPT = 4
_CE = 5000
_NCHUNK = _E // _CE


def _sc_scatter_body(bonds_h, idx_h, sums_h, cnt_h,
                     vals0, vals1, idxb0, idxb1,
                     acc0, acc1, acc2, acc3, accc,
                     sv0, sv1, si0, si1):
    c = lax.axis_index("c")
    s = lax.axis_index("s")
    b = c * (_B // _NC) + s // 4
    f0 = (s % 4) * _FPT
    vals = [vals0, vals1]
    idxb = [idxb0, idxb1]
    sv = [sv0, sv1]
    si = [si0, si1]
    accs = [acc0, acc1, acc2, acc3]

    zeros16 = jnp.zeros((16,), jnp.float32)

    def _zero_body(i, _):
        for a in accs:
            a[pl.ds(i * 16, 16)] = zeros16
        accc[pl.ds(i * 16, 16)] = zeros16
        return 0

    lax.fori_loop(0, _N // 16, _zero_body, 0)

    ones16 = jnp.ones((16,), jnp.float32)

    def _start(ch, buf):
        dv = pltpu.async_copy(
            bonds_h.at[b, pl.ds(f0, _FPT), pl.ds(ch * _CE, _CE)],
            vals[buf], sv[buf])
        di = pltpu.async_copy(
            idx_h.at[pl.ds(ch * _CE, _CE)], idxb[buf], si[buf])
        return dv, di

    def _compute(buf):
        vb = vals[buf]
        ib = idxb[buf]

        def _body(i, _):
            idx = ib[pl.ds(i * 16, 16)]
            for k in range(_FPT):
                v = vb[k, pl.ds(i * 16, 16)]
                plsc.addupdate_scatter(accs[k], [idx], v)
            plsc.addupdate_scatter(accc, [idx], ones16)
            return 0

        lax.fori_loop(0, _CE // 16, _body, 0)

    descs = [None, None]
    descs[0] = _start(0, 0)
    for ch in range(_NCHUNK):
        buf = ch & 1
        if ch + 1 < _NCHUNK:
            descs[(ch + 1) & 1] = _start(ch + 1, (ch + 1) & 1)
        dv, di = descs[buf]
        dv.wait()
        di.wait()
        _compute(buf)

    for k in range(_FPT):
        pltpu.sync_copy(accs[k], sums_h.at[b, f0 + k])

    @pl.when(jnp.logical_and(c == 0, s == 0))
    def _():
        pltpu.sync_copy(accc, cnt_h)


@functools.cache
def _sc_scatter():
    return pl.kernel(
        _sc_scatter_body,
        out_type=[
            jax.ShapeDtypeStruct((_B, _D, _N), jnp.float32),
            jax.ShapeDtypeStruct((_N,), jnp.float32),
        ],
        mesh=plsc.VectorSubcoreMesh(core_axis_name="c", subcore_axis_name="s",
                                    num_cores=_NC, num_subcores=_NS),
        compiler_params=pltpu.CompilerParams(use_tc_tiling_on_sc=False,
                                             needs_layout_passes=False),
        scratch_types=[
            pltpu.VMEM((_FPT, _CE), jnp.float32),
            pltpu.VMEM((_FPT, _CE), jnp.float32),
            pltpu.VMEM((_CE,), jnp.int32),
            pltpu.VMEM((_CE,), jnp.int32),
            pltpu.VMEM((_N,), jnp.float32),
            pltpu.VMEM((_N,), jnp.float32),
            pltpu.VMEM((_N,), jnp.float32),
            pltpu.VMEM((_N,), jnp.float32),
            pltpu.VMEM((_N,), jnp.float32),
            pltpu.SemaphoreType.DMA,
            pltpu.SemaphoreType.DMA,
            pltpu.SemaphoreType.DMA,
            pltpu.SemaphoreType.DMA,
        ],
    )


_BLK = 2048


def _mlp_body(sums_r, cnt_r, sites_r, states_r,
              w1_r, b1_r, w2_r, b2_r, w3_r, b3_r, out_r):
    b = pl.program_id(0)
    inv = 1.0 / jnp.maximum(cnt_r[...], 1.0)
    pool_t = sums_r[0] * inv

    sel = (lax.broadcasted_iota(jnp.int32, (1, _B), 1) == b)
    st_all = jnp.dot(states_r[...], w1_r[_D + _SL:, :],
                     preferred_element_type=jnp.float32)
    x_state = jnp.dot(sel.astype(jnp.float32), st_all,
                      preferred_element_type=jnp.float32)

    x_pool = lax.dot_general(pool_t, w1_r[0:_D, :],
                             (((0,), (0,)), ((), ())),
                             preferred_element_type=jnp.float32)
    h = (x_pool
         + jnp.dot(sites_r[0], w1_r[_D:_D + _SL, :],
                   preferred_element_type=jnp.float32)
         + x_state + b1_r[...])
    h = jnp.maximum(h, 0.0)
    h = jnp.maximum(jnp.dot(h, w2_r[...], preferred_element_type=jnp.float32)
                    + b2_r[...], 0.0)
    h = jnp.maximum(jnp.dot(h, w3_r[...], preferred_element_type=jnp.float32)
                    + b3_r[...], 0.0)
    out_r[0] = h


def _mlp_call(sums_t, cnt, sites, states, w1, b1, w2, b2, w3, b3):
    d_in = _D + _SL + _STL
    grid = (_B, pl.cdiv(_N, _BLK))
    return pl.pallas_call(
        _mlp_body,
        grid=grid,
        in_specs=[
            pl.BlockSpec((1, _D, _BLK), lambda b, n: (b, 0, n)),
            pl.BlockSpec((1, _BLK), lambda b, n: (0, n)),
            pl.BlockSpec((1, _BLK, _SL), lambda b, n: (b, n, 0)),
            pl.BlockSpec((_B, _STL), lambda b, n: (0, 0)),
            pl.BlockSpec((d_in, 128), lambda b, n: (0, 0)),
            pl.BlockSpec((1, 128), lambda b, n: (0, 0)),
            pl.BlockSpec((128, 128), lambda b, n: (0, 0)),
            pl.BlockSpec((1, 128), lambda b, n: (0, 0)),
            pl.BlockSpec((128, _SL), lambda b, n: (0, 0)),
            pl.BlockSpec((1, _SL), lambda b, n: (0, 0)),
        ],
        out_specs=pl.BlockSpec((1, _BLK, _SL), lambda b, n: (b, n, 0)),
        out_shape=jax.ShapeDtypeStruct((_B, _N, _SL), jnp.float32),
    )(sums_t, cnt, sites, states, w1, b1, w2, b2, w3, b3)


def kernel(sites, bonds, states, indices1, W1, b1, W2, b2, W3, b3):
    bonds_t = jnp.transpose(bonds, (0, 2, 1))
    sums_t, cnt = _sc_scatter()(bonds_t, indices1)
    return _mlp_call(sums_t, cnt.reshape(1, _N), sites, states,
                     W1, b1.reshape(1, -1), W2, b2.reshape(1, -1),
                     W3, b3.reshape(1, -1))


# speedup vs baseline: 47.6188x; 1.0114x over previous
import functools

import jax
import jax.numpy as jnp
from jax import lax
from jax.experimental import pallas as pl
from jax.experimental.pallas import tpu as pltpu
from jax.experimental.pallas import tpu_sc as plsc

_B = 8
_N = 10000
_E = 160000
_D = 16
_SL = 128
_STL = 16

_NC = 2
_NS = 16
_FPT = 4
_CE = 6400
_NCHUNK = _E // _CE
_NW = _NC * _NS


def _sc_scatter_body(bonds_h, idx_h, sums_h, cnt_h,
                     vals0, vals1, idxb0, idxb1,
                     acc0, acc1, acc2, acc3, accc,
                     sv0, sv1, si0, si1):
    c = lax.axis_index("c")
    s = lax.axis_index("s")
    b = c * (_B // _NC) + s // 4
    f0 = (s % 4) * _FPT
    vals = [vals0, vals1]
    idxb = [idxb0, idxb1]
    sv = [sv0, sv1]
    si = [si0, si1]
    accs = [acc0, acc1, acc2, acc3]

    zeros16 = jnp.zeros((16,), jnp.float32)

    def _zero_body(i, _):
        for a in accs:
            a[pl.ds(i * 16, 16)] = zeros16
        accc[pl.ds(i * 16, 16)] = zeros16
        return 0

    lax.fori_loop(0, _N // 16, _zero_body, 0)

    ones16 = jnp.ones((16,), jnp.float32)

    def _start(ch, buf):
        dv = pltpu.async_copy(
            bonds_h.at[b, pl.ds(f0, _FPT), pl.ds(ch * _CE, _CE)],
            vals[buf], sv[buf])
        di = pltpu.async_copy(
            idx_h.at[pl.ds(ch * _CE, _CE)], idxb[buf], si[buf])
        return dv, di

    w = c * _NS + s

    def _compute(buf):
        vb = vals[buf]
        ib = idxb[buf]

        def _body(j, _):
            for u in range(4):
                o = j * 64 + u * 16
                idx = ib[pl.ds(o, 16)]
                for k in range(_FPT):
                    v = vb[k, pl.ds(o, 16)]
                    plsc.addupdate_scatter(accs[k], [idx], v)
            return 0

        lax.fori_loop(0, _CE // 64, _body, 0)

    def _count(buf):
        ib = idxb[buf]

        def _body(j, _):
            for u in range(4):
                o = j * 64 + u * 16
                plsc.addupdate_scatter(accc, [ib[pl.ds(o, 16)]], ones16)
            return 0

        lax.fori_loop(0, _CE // 64, _body, 0)

    descs = [None, None]
    descs[0] = _start(0, 0)
    for ch in range(_NCHUNK):
        buf = ch & 1
        if ch + 1 < _NCHUNK:
            descs[(ch + 1) & 1] = _start(ch + 1, (ch + 1) & 1)
        dv, di = descs[buf]
        dv.wait()
        di.wait()
        _compute(buf)

        @pl.when(w == ch % _NW)
        def _():
            _count(buf)

    for k in range(_FPT):
        pltpu.sync_copy(accs[k], sums_h.at[b, f0 + k])

    pltpu.sync_copy(accc, cnt_h.at[w])


@functools.cache
def _sc_scatter():
    return pl.kernel(
        _sc_scatter_body,
        out_type=[
            jax.ShapeDtypeStruct((_B, _D, _N), jnp.float32),
            jax.ShapeDtypeStruct((_NW, _N), jnp.float32),
        ],
        mesh=plsc.VectorSubcoreMesh(core_axis_name="c", subcore_axis_name="s",
                                    num_cores=_NC, num_subcores=_NS),
        compiler_params=pltpu.CompilerParams(use_tc_tiling_on_sc=False,
                                             needs_layout_passes=False),
        scratch_types=[
            pltpu.VMEM((_FPT, _CE), jnp.float32),
            pltpu.VMEM((_FPT, _CE), jnp.float32),
            pltpu.VMEM((_CE,), jnp.int32),
            pltpu.VMEM((_CE,), jnp.int32),
            pltpu.VMEM((_N,), jnp.float32),
            pltpu.VMEM((_N,), jnp.float32),
            pltpu.VMEM((_N,), jnp.float32),
            pltpu.VMEM((_N,), jnp.float32),
            pltpu.VMEM((_N,), jnp.float32),
            pltpu.SemaphoreType.DMA,
            pltpu.SemaphoreType.DMA,
            pltpu.SemaphoreType.DMA,
            pltpu.SemaphoreType.DMA,
        ],
    )


_BLK = 2048


def _mlp_body(sums_r, cnt_r, sites_r, states_r,
              w1_r, b1_r, w2_r, b2_r, w3_r, b3_r, out_r):
    b = pl.program_id(0)
    cnt = jnp.sum(cnt_r[...], axis=0, keepdims=True)
    inv = 1.0 / jnp.maximum(cnt, 1.0)
    pool_t = sums_r[0] * inv

    sel = (lax.broadcasted_iota(jnp.int32, (1, _B), 1) == b)
    st_all = jnp.dot(states_r[...], w1_r[_D + _SL:, :],
                     preferred_element_type=jnp.float32)
    x_state = jnp.dot(sel.astype(jnp.float32), st_all,
                      preferred_element_type=jnp.float32)

    x_pool = lax.dot_general(pool_t, w1_r[0:_D, :],
                             (((0,), (0,)), ((), ())),
                             preferred_element_type=jnp.float32)
    h = (x_pool
         + jnp.dot(sites_r[0], w1_r[_D:_D + _SL, :],
                   preferred_element_type=jnp.float32)
         + x_state + b1_r[...])
    h = jnp.maximum(h, 0.0)
    h = jnp.maximum(jnp.dot(h, w2_r[...], preferred_element_type=jnp.float32)
                    + b2_r[...], 0.0)
    h = jnp.maximum(jnp.dot(h, w3_r[...], preferred_element_type=jnp.float32)
                    + b3_r[...], 0.0)
    out_r[0] = h


def _mlp_call(sums_t, cnt, sites, states, w1, b1, w2, b2, w3, b3):
    d_in = _D + _SL + _STL
    grid = (_B, pl.cdiv(_N, _BLK))
    return pl.pallas_call(
        _mlp_body,
        grid=grid,
        in_specs=[
            pl.BlockSpec((1, _D, _BLK), lambda b, n: (b, 0, n)),
            pl.BlockSpec((_NW, _BLK), lambda b, n: (0, n)),
            pl.BlockSpec((1, _BLK, _SL), lambda b, n: (b, n, 0)),
            pl.BlockSpec((_B, _STL), lambda b, n: (0, 0)),
            pl.BlockSpec((d_in, 128), lambda b, n: (0, 0)),
            pl.BlockSpec((1, 128), lambda b, n: (0, 0)),
            pl.BlockSpec((128, 128), lambda b, n: (0, 0)),
            pl.BlockSpec((1, 128), lambda b, n: (0, 0)),
            pl.BlockSpec((128, _SL), lambda b, n: (0, 0)),
            pl.BlockSpec((1, _SL), lambda b, n: (0, 0)),
        ],
        out_specs=pl.BlockSpec((1, _BLK, _SL), lambda b, n: (b, n, 0)),
        out_shape=jax.ShapeDtypeStruct((_B, _N, _SL), jnp.float32),
    )(sums_t, cnt, sites, states, w1, b1, w2, b2, w3, b3)


def kernel(sites, bonds, states, indices1, W1, b1, W2, b2, W3, b3):
    bonds_t = jnp.transpose(bonds, (0, 2, 1))
    sums_t, cnt = _sc_scatter()(bonds_t, indices1)
    return _mlp_call(sums_t, cnt, sites, states,
                     W1, b1.reshape(1, -1), W2, b2.reshape(1, -1),
                     W3, b3.reshape(1, -1))


# speedup vs baseline: 60.7644x; 1.2761x over previous
import functools

import jax
import jax.numpy as jnp
from jax import lax
from jax.experimental import pallas as pl
from jax.experimental.pallas import tpu as pltpu
from jax.experimental.pallas import tpu_sc as plsc

_B = 8
_N = 10000
_E = 160000
_D = 16
_SL = 128
_STL = 16

_NC = 2
_NS = 16
_FPT = 4
_CE = 3200
_NCHUNK = _E // _CE
_NW = _NC * _NS


def _sc_scatter_body(bonds_h, idx_h, sums_h, cnt_h,
                     vals0, vals1, idxb0, idxb1,
                     acc0, acc1, acc2, acc3, accc,
                     sv0, sv1, si0, si1):
    c = lax.axis_index("c")
    s = lax.axis_index("s")
    b = c * (_B // _NC) + s // 4
    f0 = (s % 4) * _FPT
    vals = [vals0, vals1]
    idxb = [idxb0, idxb1]
    sv = [sv0, sv1]
    si = [si0, si1]
    accs = [acc0, acc1, acc2, acc3]

    zeros16 = jnp.zeros((16,), jnp.float32)

    def _zero_body(i, _):
        for a in accs:
            a[pl.ds(i * 16, 16)] = zeros16
        accc[pl.ds(i * 16, 16)] = zeros16
        return 0

    lax.fori_loop(0, _N // 16, _zero_body, 0)

    ones16 = jnp.ones((16,), jnp.float32)

    def _start(ch, buf):
        dv = pltpu.async_copy(
            bonds_h.at[b, pl.ds(f0, _FPT), pl.ds(ch * _CE, _CE)],
            vals[buf], sv[buf])
        di = pltpu.async_copy(
            idx_h.at[pl.ds(ch * _CE, _CE)], idxb[buf], si[buf])
        return dv, di

    w = c * _NS + s

    def _compute(buf):
        vb = vals[buf]
        ib = idxb[buf]

        def _body(j, _):
            for u in range(4):
                o = j * 64 + u * 16
                idx = ib[pl.ds(o, 16)]
                for k in range(_FPT):
                    v = vb[k, pl.ds(o, 16)]
                    plsc.addupdate_scatter(accs[k], [idx], v)
            return 0

        lax.fori_loop(0, _CE // 64, _body, 0)

    def _count(buf):
        ib = idxb[buf]

        def _body(j, _):
            for u in range(4):
                o = j * 64 + u * 16
                plsc.addupdate_scatter(accc, [ib[pl.ds(o, 16)]], ones16)
            return 0

        lax.fori_loop(0, _CE // 64, _body, 0)

    descs = [None, None]
    descs[0] = _start(0, 0)
    for ch in range(_NCHUNK):
        buf = ch & 1
        if ch + 1 < _NCHUNK:
            descs[(ch + 1) & 1] = _start(ch + 1, (ch + 1) & 1)
        dv, di = descs[buf]
        dv.wait()
        di.wait()
        _compute(buf)

        @pl.when(w == ch % _NW)
        def _():
            _count(buf)

    for k in range(_FPT):
        pltpu.sync_copy(accs[k], sums_h.at[b, f0 + k])

    pltpu.sync_copy(accc, cnt_h.at[w])


@functools.cache
def _sc_scatter():
    return pl.kernel(
        _sc_scatter_body,
        out_type=[
            jax.ShapeDtypeStruct((_B, _D, _N), jnp.float32),
            jax.ShapeDtypeStruct((_NW, _N), jnp.float32),
        ],
        mesh=plsc.VectorSubcoreMesh(core_axis_name="c", subcore_axis_name="s",
                                    num_cores=_NC, num_subcores=_NS),
        compiler_params=pltpu.CompilerParams(use_tc_tiling_on_sc=True,
                                             needs_layout_passes=False),
        scratch_types=[
            pltpu.VMEM((_FPT, _CE), jnp.float32),
            pltpu.VMEM((_FPT, _CE), jnp.float32),
            pltpu.VMEM((_CE,), jnp.int32),
            pltpu.VMEM((_CE,), jnp.int32),
            pltpu.VMEM((_N,), jnp.float32),
            pltpu.VMEM((_N,), jnp.float32),
            pltpu.VMEM((_N,), jnp.float32),
            pltpu.VMEM((_N,), jnp.float32),
            pltpu.VMEM((_N,), jnp.float32),
            pltpu.SemaphoreType.DMA,
            pltpu.SemaphoreType.DMA,
            pltpu.SemaphoreType.DMA,
            pltpu.SemaphoreType.DMA,
        ],
    )


_BLK = 2048


def _mlp_body(sums_r, cnt_r, sites_r, states_r,
              w1_r, b1_r, w2_r, b2_r, w3_r, b3_r, out_r):
    b = pl.program_id(0)
    cnt = jnp.sum(cnt_r[...], axis=0, keepdims=True)
    inv = 1.0 / jnp.maximum(cnt, 1.0)
    pool_t = sums_r[0] * inv

    sel = (lax.broadcasted_iota(jnp.int32, (1, _B), 1) == b)
    st_all = jnp.dot(states_r[...], w1_r[_D + _SL:, :],
                     preferred_element_type=jnp.float32)
    x_state = jnp.dot(sel.astype(jnp.float32), st_all,
                      preferred_element_type=jnp.float32)

    x_pool = lax.dot_general(pool_t, w1_r[0:_D, :],
                             (((0,), (0,)), ((), ())),
                             preferred_element_type=jnp.float32)
    h = (x_pool
         + jnp.dot(sites_r[0], w1_r[_D:_D + _SL, :],
                   preferred_element_type=jnp.float32)
         + x_state + b1_r[...])
    h = jnp.maximum(h, 0.0)
    h = jnp.maximum(jnp.dot(h, w2_r[...], preferred_element_type=jnp.float32)
                    + b2_r[...], 0.0)
    h = jnp.maximum(jnp.dot(h, w3_r[...], preferred_element_type=jnp.float32)
                    + b3_r[...], 0.0)
    out_r[0] = h


def _mlp_call(sums_t, cnt, sites, states, w1, b1, w2, b2, w3, b3):
    d_in = _D + _SL + _STL
    grid = (_B, pl.cdiv(_N, _BLK))
    return pl.pallas_call(
        _mlp_body,
        grid=grid,
        in_specs=[
            pl.BlockSpec((1, _D, _BLK), lambda b, n: (b, 0, n)),
            pl.BlockSpec((_NW, _BLK), lambda b, n: (0, n)),
            pl.BlockSpec((1, _BLK, _SL), lambda b, n: (b, n, 0)),
            pl.BlockSpec((_B, _STL), lambda b, n: (0, 0)),
            pl.BlockSpec((d_in, 128), lambda b, n: (0, 0)),
            pl.BlockSpec((1, 128), lambda b, n: (0, 0)),
            pl.BlockSpec((128, 128), lambda b, n: (0, 0)),
            pl.BlockSpec((1, 128), lambda b, n: (0, 0)),
            pl.BlockSpec((128, _SL), lambda b, n: (0, 0)),
            pl.BlockSpec((1, _SL), lambda b, n: (0, 0)),
        ],
        out_specs=pl.BlockSpec((1, _BLK, _SL), lambda b, n: (b, n, 0)),
        out_shape=jax.ShapeDtypeStruct((_B, _N, _SL), jnp.float32),
    )(sums_t, cnt, sites, states, w1, b1, w2, b2, w3, b3)


def kernel(sites, bonds, states, indices1, W1, b1, W2, b2, W3, b3):
    bonds_t = jnp.transpose(bonds, (0, 2, 1))
    sums_t, cnt = _sc_scatter()(bonds_t, indices1)
    return _mlp_call(sums_t, cnt, sites, states,
                     W1, b1.reshape(1, -1), W2, b2.reshape(1, -1),
                     W3, b3.reshape(1, -1))


# speedup vs baseline: 62.1651x; 1.0231x over previous
import functools

import jax
import jax.numpy as jnp
from jax import lax
from jax.experimental import pallas as pl
from jax.experimental.pallas import tpu as pltpu
from jax.experimental.pallas import tpu_sc as plsc

_B = 8
_N = 10000
_E = 160000
_D = 16
_SL = 128
_STL = 16

_NC = 2
_NS = 16
_FPT = 4
_CE = 3200
_NCHUNK = _E // _CE
_NW = _NC * _NS


def _sc_scatter_body(bonds_h, idx_h, sums_h, cnt_h,
                     vals0, vals1, idxb0, idxb1,
                     acc0, acc1, acc2, acc3, accc,
                     sv0, sv1, si0, si1):
    c = lax.axis_index("c")
    s = lax.axis_index("s")
    b = c * (_B // _NC) + s // 4
    f0 = (s % 4) * _FPT
    vals = [vals0, vals1]
    idxb = [idxb0, idxb1]
    sv = [sv0, sv1]
    si = [si0, si1]
    accs = [acc0, acc1, acc2, acc3]

    zeros16 = jnp.zeros((16,), jnp.float32)

    def _zero_body(i, _):
        for a in accs:
            a[pl.ds(i * 16, 16)] = zeros16
        accc[pl.ds(i * 16, 16)] = zeros16
        return 0

    lax.fori_loop(0, _N // 16, _zero_body, 0)

    ones16 = jnp.ones((16,), jnp.float32)

    def _start(ch, buf):
        dv = pltpu.async_copy(
            bonds_h.at[b, pl.ds(f0, _FPT), pl.ds(ch * _CE, _CE)],
            vals[buf], sv[buf])
        di = pltpu.async_copy(
            idx_h.at[pl.ds(ch * _CE, _CE)], idxb[buf], si[buf])
        return dv, di

    w = c * _NS + s

    def _compute(buf):
        vb = vals[buf]
        ib = idxb[buf]

        def _body(j, _):
            for u in range(4):
                o = j * 64 + u * 16
                idx = ib[pl.ds(o, 16)]
                for k in range(_FPT):
                    v = vb[k, pl.ds(o, 16)]
                    plsc.addupdate_scatter(accs[k], [idx], v)
            return 0

        lax.fori_loop(0, _CE // 64, _body, 0)

    def _count(buf):
        ib = idxb[buf]

        def _body(j, _):
            for u in range(4):
                o = j * 64 + u * 16
                plsc.addupdate_scatter(accc, [ib[pl.ds(o, 16)]], ones16)
            return 0

        lax.fori_loop(0, _CE // 64, _body, 0)

    descs = [None, None]
    descs[0] = _start(0, 0)
    for ch in range(_NCHUNK):
        buf = ch & 1
        if ch + 1 < _NCHUNK:
            descs[(ch + 1) & 1] = _start(ch + 1, (ch + 1) & 1)
        dv, di = descs[buf]
        dv.wait()
        di.wait()
        _compute(buf)

        @pl.when(w == ch % _NW)
        def _():
            _count(buf)

    for k in range(_FPT):
        pltpu.sync_copy(accs[k], sums_h.at[b, f0 + k])

    pltpu.sync_copy(accc, cnt_h.at[w])


@functools.cache
def _sc_scatter():
    return pl.kernel(
        _sc_scatter_body,
        out_type=[
            jax.ShapeDtypeStruct((_B, _D, _N), jnp.float32),
            jax.ShapeDtypeStruct((_NW, _N), jnp.float32),
        ],
        mesh=plsc.VectorSubcoreMesh(core_axis_name="c", subcore_axis_name="s",
                                    num_cores=_NC, num_subcores=_NS),
        compiler_params=pltpu.CompilerParams(use_tc_tiling_on_sc=True,
                                             needs_layout_passes=False),
        scratch_types=[
            pltpu.VMEM((_FPT, _CE), jnp.float32),
            pltpu.VMEM((_FPT, _CE), jnp.float32),
            pltpu.VMEM((_CE,), jnp.int32),
            pltpu.VMEM((_CE,), jnp.int32),
            pltpu.VMEM((_N,), jnp.float32),
            pltpu.VMEM((_N,), jnp.float32),
            pltpu.VMEM((_N,), jnp.float32),
            pltpu.VMEM((_N,), jnp.float32),
            pltpu.VMEM((_N,), jnp.float32),
            pltpu.SemaphoreType.DMA,
            pltpu.SemaphoreType.DMA,
            pltpu.SemaphoreType.DMA,
            pltpu.SemaphoreType.DMA,
        ],
    )


_BLK = 4096


def _mlp_body(sums_r, cnt_r, sites_r, states_r,
              w1_r, b1_r, w2_r, b2_r, w3_r, b3_r, out_r):
    b = pl.program_id(0)
    cnt = jnp.sum(cnt_r[...], axis=0, keepdims=True)
    inv = 1.0 / jnp.maximum(cnt, 1.0)
    pool_t = sums_r[0] * inv

    sel = (lax.broadcasted_iota(jnp.int32, (1, _B), 1) == b)
    st_all = jnp.dot(states_r[...], w1_r[_D + _SL:, :],
                     preferred_element_type=jnp.float32)
    x_state = jnp.dot(sel.astype(jnp.float32), st_all,
                      preferred_element_type=jnp.float32)

    x_pool = lax.dot_general(pool_t, w1_r[0:_D, :],
                             (((0,), (0,)), ((), ())),
                             preferred_element_type=jnp.float32)
    h = (x_pool
         + jnp.dot(sites_r[0], w1_r[_D:_D + _SL, :],
                   preferred_element_type=jnp.float32)
         + x_state + b1_r[...])
    h = jnp.maximum(h, 0.0)
    h = jnp.maximum(jnp.dot(h, w2_r[...], preferred_element_type=jnp.float32)
                    + b2_r[...], 0.0)
    h = jnp.maximum(jnp.dot(h, w3_r[...], preferred_element_type=jnp.float32)
                    + b3_r[...], 0.0)
    out_r[0] = h


def _mlp_call(sums_t, cnt, sites, states, w1, b1, w2, b2, w3, b3):
    d_in = _D + _SL + _STL
    grid = (_B, pl.cdiv(_N, _BLK))
    return pl.pallas_call(
        _mlp_body,
        grid=grid,
        in_specs=[
            pl.BlockSpec((1, _D, _BLK), lambda b, n: (b, 0, n)),
            pl.BlockSpec((_NW, _BLK), lambda b, n: (0, n)),
            pl.BlockSpec((1, _BLK, _SL), lambda b, n: (b, n, 0)),
            pl.BlockSpec((_B, _STL), lambda b, n: (0, 0)),
            pl.BlockSpec((d_in, 128), lambda b, n: (0, 0)),
            pl.BlockSpec((1, 128), lambda b, n: (0, 0)),
            pl.BlockSpec((128, 128), lambda b, n: (0, 0)),
            pl.BlockSpec((1, 128), lambda b, n: (0, 0)),
            pl.BlockSpec((128, _SL), lambda b, n: (0, 0)),
            pl.BlockSpec((1, _SL), lambda b, n: (0, 0)),
        ],
        out_specs=pl.BlockSpec((1, _BLK, _SL), lambda b, n: (b, n, 0)),
        out_shape=jax.ShapeDtypeStruct((_B, _N, _SL), jnp.float32),
    )(sums_t, cnt, sites, states, w1, b1, w2, b2, w3, b3)


def kernel(sites, bonds, states, indices1, W1, b1, W2, b2, W3, b3):
    bonds_t = jnp.transpose(bonds, (0, 2, 1))
    sums_t, cnt = _sc_scatter()(bonds_t, indices1)
    return _mlp_call(sums_t, cnt, sites, states,
                     W1, b1.reshape(1, -1), W2, b2.reshape(1, -1),
                     W3, b3.reshape(1, -1))


# speedup vs baseline: 64.3506x; 1.0352x over previous
import functools

import jax
import jax.numpy as jnp
from jax import lax
from jax.experimental import pallas as pl
from jax.experimental.pallas import tpu as pltpu
from jax.experimental.pallas import tpu_sc as plsc

_B = 8
_N = 10000
_E = 160000
_D = 16
_SL = 128
_STL = 16

_NC = 2
_NS = 16
_FPT = 4
_CE = 3200
_NCHUNK = _E // _CE
_NW = _NC * _NS


def _sc_scatter_body(bonds_h, idx_h, sums_h, cnt_h,
                     vals0, vals1, idxb0, idxb1,
                     acc0, acc1, acc2, acc3, accc,
                     sv0, sv1, si0, si1):
    c = lax.axis_index("c")
    s = lax.axis_index("s")
    b = c * (_B // _NC) + s // 4
    f0 = (s % 4) * _FPT
    vals = [vals0, vals1]
    idxb = [idxb0, idxb1]
    sv = [sv0, sv1]
    si = [si0, si1]
    accs = [acc0, acc1, acc2, acc3]

    zeros16 = jnp.zeros((16,), jnp.float32)

    def _zero_body(i, _):
        for a in accs:
            a[pl.ds(i * 16, 16)] = zeros16
        accc[pl.ds(i * 16, 16)] = zeros16
        return 0

    lax.fori_loop(0, _N // 16, _zero_body, 0)

    ones16 = jnp.ones((16,), jnp.float32)

    def _start(ch, buf):
        dv = pltpu.async_copy(
            bonds_h.at[b, pl.ds(f0, _FPT), pl.ds(ch * _CE, _CE)],
            vals[buf], sv[buf])
        di = pltpu.async_copy(
            idx_h.at[pl.ds(ch * _CE, _CE)], idxb[buf], si[buf])
        return dv, di

    w = c * _NS + s

    def _compute(buf):
        vb = vals[buf]
        ib = idxb[buf]

        def _body(j, _):
            for u in range(4):
                o = j * 64 + u * 16
                idx = ib[pl.ds(o, 16)]
                for k in range(_FPT):
                    v = vb[k, pl.ds(o, 16)]
                    plsc.addupdate_scatter(accs[k], [idx], v)
            return 0

        lax.fori_loop(0, _CE // 64, _body, 0)

    def _count(buf):
        ib = idxb[buf]

        def _body(j, _):
            for u in range(4):
                o = j * 64 + u * 16
                plsc.addupdate_scatter(accc, [ib[pl.ds(o, 16)]], ones16)
            return 0

        lax.fori_loop(0, _CE // 64, _body, 0)

    descs = [None, None]
    descs[0] = _start(0, 0)
    for ch in range(_NCHUNK):
        buf = ch & 1
        if ch + 1 < _NCHUNK:
            descs[(ch + 1) & 1] = _start(ch + 1, (ch + 1) & 1)
        dv, di = descs[buf]
        dv.wait()
        di.wait()
        _compute(buf)

        @pl.when(w == ch % _NW)
        def _():
            _count(buf)

    for k in range(_FPT):
        pltpu.sync_copy(accs[k], sums_h.at[b, f0 + k])

    pltpu.sync_copy(accc, cnt_h.at[w])


@functools.cache
def _sc_scatter():
    return pl.kernel(
        _sc_scatter_body,
        out_type=[
            jax.ShapeDtypeStruct((_B, _D, _N), jnp.float32),
            jax.ShapeDtypeStruct((_NW, _N), jnp.float32),
        ],
        mesh=plsc.VectorSubcoreMesh(core_axis_name="c", subcore_axis_name="s",
                                    num_cores=_NC, num_subcores=_NS),
        compiler_params=pltpu.CompilerParams(use_tc_tiling_on_sc=True,
                                             needs_layout_passes=False),
        scratch_types=[
            pltpu.VMEM((_FPT, _CE), jnp.float32),
            pltpu.VMEM((_FPT, _CE), jnp.float32),
            pltpu.VMEM((_CE,), jnp.int32),
            pltpu.VMEM((_CE,), jnp.int32),
            pltpu.VMEM((_N,), jnp.float32),
            pltpu.VMEM((_N,), jnp.float32),
            pltpu.VMEM((_N,), jnp.float32),
            pltpu.VMEM((_N,), jnp.float32),
            pltpu.VMEM((_N,), jnp.float32),
            pltpu.SemaphoreType.DMA,
            pltpu.SemaphoreType.DMA,
            pltpu.SemaphoreType.DMA,
            pltpu.SemaphoreType.DMA,
        ],
    )


_BLK = 5120


def _mlp_body(sums_r, cnt_r, sites_r, states_r,
              w1_r, b1_r, w2_r, b2_r, w3_r, b3_r, out_r):
    b = pl.program_id(0)
    cnt = jnp.sum(cnt_r[...], axis=0, keepdims=True)
    inv = 1.0 / jnp.maximum(cnt, 1.0)
    pool_t = sums_r[0] * inv

    sel = (lax.broadcasted_iota(jnp.int32, (1, _B), 1) == b)
    st_all = jnp.dot(states_r[...], w1_r[_D + _SL:, :],
                     preferred_element_type=jnp.float32)
    x_state = jnp.dot(sel.astype(jnp.float32), st_all,
                      preferred_element_type=jnp.float32)

    x_pool = lax.dot_general(pool_t, w1_r[0:_D, :],
                             (((0,), (0,)), ((), ())),
                             preferred_element_type=jnp.float32)
    h = (x_pool
         + jnp.dot(sites_r[0], w1_r[_D:_D + _SL, :],
                   preferred_element_type=jnp.float32)
         + x_state + b1_r[...])
    h = jnp.maximum(h, 0.0)
    h = jnp.maximum(jnp.dot(h, w2_r[...], preferred_element_type=jnp.float32)
                    + b2_r[...], 0.0)
    h = jnp.maximum(jnp.dot(h, w3_r[...], preferred_element_type=jnp.float32)
                    + b3_r[...], 0.0)
    out_r[0] = h


def _mlp_call(sums_t, cnt, sites, states, w1, b1, w2, b2, w3, b3):
    d_in = _D + _SL + _STL
    grid = (_B, pl.cdiv(_N, _BLK))
    return pl.pallas_call(
        _mlp_body,
        grid=grid,
        in_specs=[
            pl.BlockSpec((1, _D, _BLK), lambda b, n: (b, 0, n)),
            pl.BlockSpec((_NW, _BLK), lambda b, n: (0, n)),
            pl.BlockSpec((1, _BLK, _SL), lambda b, n: (b, n, 0)),
            pl.BlockSpec((_B, _STL), lambda b, n: (0, 0)),
            pl.BlockSpec((d_in, 128), lambda b, n: (0, 0)),
            pl.BlockSpec((1, 128), lambda b, n: (0, 0)),
            pl.BlockSpec((128, 128), lambda b, n: (0, 0)),
            pl.BlockSpec((1, 128), lambda b, n: (0, 0)),
            pl.BlockSpec((128, _SL), lambda b, n: (0, 0)),
            pl.BlockSpec((1, _SL), lambda b, n: (0, 0)),
        ],
        out_specs=pl.BlockSpec((1, _BLK, _SL), lambda b, n: (b, n, 0)),
        out_shape=jax.ShapeDtypeStruct((_B, _N, _SL), jnp.float32),
    )(sums_t, cnt, sites, states, w1, b1, w2, b2, w3, b3)


def kernel(sites, bonds, states, indices1, W1, b1, W2, b2, W3, b3):
    bonds_t = jnp.transpose(bonds, (0, 2, 1))
    sums_t, cnt = _sc_scatter()(bonds_t, indices1)
    return _mlp_call(sums_t, cnt, sites, states,
                     W1, b1.reshape(1, -1), W2, b2.reshape(1, -1),
                     W3, b3.reshape(1, -1))


# speedup vs baseline: 65.8578x; 1.0234x over previous
import functools

import jax
import jax.numpy as jnp
from jax import lax
from jax.experimental import pallas as pl
from jax.experimental.pallas import tpu as pltpu
from jax.experimental.pallas import tpu_sc as plsc

_B = 8
_N = 10000
_E = 160000
_D = 16
_SL = 128
_STL = 16

_NC = 2
_NS = 16
_FPT = 4
_CE = 3200
_NCHUNK = _E // _CE
_NW = _NC * _NS


def _sc_scatter_body(bonds_h, idx_h, sums_h, cnt_h,
                     vals0, vals1, idxb0, idxb1,
                     acc0, acc1, acc2, acc3, accc,
                     sv0, sv1, si0, si1):
    c = lax.axis_index("c")
    s = lax.axis_index("s")
    b = c * (_B // _NC) + s // 4
    f0 = (s % 4) * _FPT
    vals = [vals0, vals1]
    idxb = [idxb0, idxb1]
    sv = [sv0, sv1]
    si = [si0, si1]
    accs = [acc0, acc1, acc2, acc3]

    zeros16 = jnp.zeros((16,), jnp.float32)

    def _zero_body(i, _):
        for a in accs:
            a[pl.ds(i * 16, 16)] = zeros16
        accc[pl.ds(i * 16, 16)] = zeros16
        return 0

    lax.fori_loop(0, _N // 16, _zero_body, 0)

    ones16 = jnp.ones((16,), jnp.float32)

    def _start(ch, buf):
        dv = pltpu.async_copy(
            bonds_h.at[b, pl.ds(f0, _FPT), pl.ds(ch * _CE, _CE)],
            vals[buf], sv[buf])
        di = pltpu.async_copy(
            idx_h.at[pl.ds(ch * _CE, _CE)], idxb[buf], si[buf])
        return dv, di

    w = c * _NS + s

    def _compute(buf):
        vb = vals[buf]
        ib = idxb[buf]

        def _body(j, _):
            for u in range(4):
                o = j * 64 + u * 16
                idx = ib[pl.ds(o, 16)]
                for k in range(_FPT):
                    v = vb[k, pl.ds(o, 16)]
                    plsc.addupdate_scatter(accs[k], [idx], v)
            return 0

        lax.fori_loop(0, _CE // 64, _body, 0)

    def _count(buf):
        ib = idxb[buf]

        def _body(j, _):
            for u in range(4):
                o = j * 64 + u * 16
                plsc.addupdate_scatter(accc, [ib[pl.ds(o, 16)]], ones16)
            return 0

        lax.fori_loop(0, _CE // 64, _body, 0)

    descs = [None, None]
    descs[0] = _start(0, 0)
    for ch in range(_NCHUNK):
        buf = ch & 1
        if ch + 1 < _NCHUNK:
            descs[(ch + 1) & 1] = _start(ch + 1, (ch + 1) & 1)
        dv, di = descs[buf]
        dv.wait()
        di.wait()
        _compute(buf)

        @pl.when(w == ch % _NW)
        def _():
            _count(buf)

    for k in range(_FPT):
        pltpu.sync_copy(accs[k], sums_h.at[b, f0 + k])

    pltpu.sync_copy(accc, cnt_h.at[w])


@functools.cache
def _sc_scatter():
    return pl.kernel(
        _sc_scatter_body,
        out_type=[
            jax.ShapeDtypeStruct((_B, _D, _N), jnp.float32),
            jax.ShapeDtypeStruct((_NW, _N), jnp.float32),
        ],
        mesh=plsc.VectorSubcoreMesh(core_axis_name="c", subcore_axis_name="s",
                                    num_cores=_NC, num_subcores=_NS),
        compiler_params=pltpu.CompilerParams(use_tc_tiling_on_sc=True,
                                             needs_layout_passes=False),
        scratch_types=[
            pltpu.VMEM((_FPT, _CE), jnp.float32),
            pltpu.VMEM((_FPT, _CE), jnp.float32),
            pltpu.VMEM((_CE,), jnp.int32),
            pltpu.VMEM((_CE,), jnp.int32),
            pltpu.VMEM((_N,), jnp.float32),
            pltpu.VMEM((_N,), jnp.float32),
            pltpu.VMEM((_N,), jnp.float32),
            pltpu.VMEM((_N,), jnp.float32),
            pltpu.VMEM((_N,), jnp.float32),
            pltpu.SemaphoreType.DMA,
            pltpu.SemaphoreType.DMA,
            pltpu.SemaphoreType.DMA,
            pltpu.SemaphoreType.DMA,
        ],
    )


_BLK = 10000


def _mlp_body(sums_r, cnt_r, sites_r, states_r,
              w1_r, b1_r, w2_r, b2_r, w3_r, b3_r, out_r):
    b = pl.program_id(0)
    cnt = jnp.sum(cnt_r[...], axis=0, keepdims=True)
    inv = 1.0 / jnp.maximum(cnt, 1.0)
    pool_t = sums_r[0] * inv

    sel = (lax.broadcasted_iota(jnp.int32, (1, _B), 1) == b)
    st_all = jnp.dot(states_r[...], w1_r[_D + _SL:, :],
                     preferred_element_type=jnp.float32)
    x_state = jnp.dot(sel.astype(jnp.float32), st_all,
                      preferred_element_type=jnp.float32)

    x_pool = lax.dot_general(pool_t, w1_r[0:_D, :],
                             (((0,), (0,)), ((), ())),
                             preferred_element_type=jnp.float32)
    h = (x_pool
         + jnp.dot(sites_r[0], w1_r[_D:_D + _SL, :],
                   preferred_element_type=jnp.float32)
         + x_state + b1_r[...])
    h = jnp.maximum(h, 0.0)
    h = jnp.maximum(jnp.dot(h, w2_r[...], preferred_element_type=jnp.float32)
                    + b2_r[...], 0.0)
    h = jnp.maximum(jnp.dot(h, w3_r[...], preferred_element_type=jnp.float32)
                    + b3_r[...], 0.0)
    out_r[0] = h


def _mlp_call(sums_t, cnt, sites, states, w1, b1, w2, b2, w3, b3):
    d_in = _D + _SL + _STL
    grid = (_B, pl.cdiv(_N, _BLK))
    return pl.pallas_call(
        _mlp_body,
        grid=grid,
        in_specs=[
            pl.BlockSpec((1, _D, _BLK), lambda b, n: (b, 0, n)),
            pl.BlockSpec((_NW, _BLK), lambda b, n: (0, n)),
            pl.BlockSpec((1, _BLK, _SL), lambda b, n: (b, n, 0)),
            pl.BlockSpec((_B, _STL), lambda b, n: (0, 0)),
            pl.BlockSpec((d_in, 128), lambda b, n: (0, 0)),
            pl.BlockSpec((1, 128), lambda b, n: (0, 0)),
            pl.BlockSpec((128, 128), lambda b, n: (0, 0)),
            pl.BlockSpec((1, 128), lambda b, n: (0, 0)),
            pl.BlockSpec((128, _SL), lambda b, n: (0, 0)),
            pl.BlockSpec((1, _SL), lambda b, n: (0, 0)),
        ],
        out_specs=pl.BlockSpec((1, _BLK, _SL), lambda b, n: (b, n, 0)),
        out_shape=jax.ShapeDtypeStruct((_B, _N, _SL), jnp.float32),
    )(sums_t, cnt, sites, states, w1, b1, w2, b2, w3, b3)


def kernel(sites, bonds, states, indices1, W1, b1, W2, b2, W3, b3):
    bonds_t = jnp.transpose(bonds, (0, 2, 1))
    sums_t, cnt = _sc_scatter()(bonds_t, indices1)
    return _mlp_call(sums_t, cnt, sites, states,
                     W1, b1.reshape(1, -1), W2, b2.reshape(1, -1),
                     W3, b3.reshape(1, -1))


# speedup vs baseline: 105.7183x; 1.6053x over previous
import functools

import jax
import jax.numpy as jnp
from jax import lax
from jax.experimental import pallas as pl
from jax.experimental.pallas import tpu as pltpu
from jax.experimental.pallas import tpu_sc as plsc

_B = 8
_N = 10000
_E = 160000
_D = 16
_SL = 128
_STL = 16

_NC = 2
_NS = 16
_FPT = 4
_CE = 3200
_NCHUNK = _E // _CE
_NW = _NC * _NS


def _sc_scatter_body(bonds_h, idx_h, sums_h, cnt_h,
                     vals0, vals1, idxb0, idxb1,
                     acc0, acc1, acc2, acc3, accc,
                     sv0, sv1, si0, si1):
    c = lax.axis_index("c")
    s = lax.axis_index("s")
    b = c * (_B // _NC) + s // 4
    f0 = (s % 4) * _FPT
    vals = [vals0, vals1]
    idxb = [idxb0, idxb1]
    sv = [sv0, sv1]
    si = [si0, si1]
    accs = [acc0, acc1, acc2, acc3]

    zeros16 = jnp.zeros((16,), jnp.float32)

    def _zero_body(i, _):
        for a in accs:
            a[pl.ds(i * 16, 16)] = zeros16
        accc[pl.ds(i * 16, 16)] = zeros16
        return 0

    lax.fori_loop(0, _N // 16, _zero_body, 0)

    ones16 = jnp.ones((16,), jnp.float32)

    def _start(ch, buf):
        dv = pltpu.async_copy(
            bonds_h.at[b, pl.ds(f0, _FPT), pl.ds(ch * _CE, _CE)],
            vals[buf], sv[buf])
        di = pltpu.async_copy(
            idx_h.at[pl.ds(ch * _CE, _CE)], idxb[buf], si[buf])
        return dv, di

    w = c * _NS + s

    def _compute(buf):
        vb = vals[buf]
        ib = idxb[buf]

        def _load(j):
            g = []
            for u in range(4):
                o = j * 64 + u * 16
                idx = ib[pl.ds(o, 16)]
                vs = tuple(vb[k, pl.ds(o, 16)] for k in range(_FPT))
                g.append((idx, vs))
            return tuple(g)

        def _scatter(g):
            for idx, vs in g:
                for k in range(_FPT):
                    plsc.addupdate_scatter(accs[k], [idx], vs[k])

        def _body(j, carry):
            nxt = _load(j + 1)
            _scatter(carry)
            return nxt

        last = lax.fori_loop(0, _CE // 64 - 1, _body, _load(0))
        _scatter(last)

    def _count(buf):
        ib = idxb[buf]

        def _body(j, _):
            for u in range(4):
                o = j * 64 + u * 16
                plsc.addupdate_scatter(accc, [ib[pl.ds(o, 16)]], ones16)
            return 0

        lax.fori_loop(0, _CE // 64, _body, 0)

    descs = [None, None]
    descs[0] = _start(0, 0)
    for ch in range(_NCHUNK):
        buf = ch & 1
        if ch + 1 < _NCHUNK:
            descs[(ch + 1) & 1] = _start(ch + 1, (ch + 1) & 1)
        dv, di = descs[buf]
        dv.wait()
        di.wait()
        _compute(buf)

        @pl.when(w == ch % _NW)
        def _():
            _count(buf)

    for k in range(_FPT):
        pltpu.sync_copy(accs[k], sums_h.at[b, f0 + k])

    pltpu.sync_copy(accc, cnt_h.at[w])


@functools.cache
def _sc_scatter():
    return pl.kernel(
        _sc_scatter_body,
        out_type=[
            jax.ShapeDtypeStruct((_B, _D, _N), jnp.float32),
            jax.ShapeDtypeStruct((_NW, _N), jnp.float32),
        ],
        mesh=plsc.VectorSubcoreMesh(core_axis_name="c", subcore_axis_name="s",
                                    num_cores=_NC, num_subcores=_NS),
        compiler_params=pltpu.CompilerParams(use_tc_tiling_on_sc=True,
                                             needs_layout_passes=False),
        scratch_types=[
            pltpu.VMEM((_FPT, _CE), jnp.float32),
            pltpu.VMEM((_FPT, _CE), jnp.float32),
            pltpu.VMEM((_CE,), jnp.int32),
            pltpu.VMEM((_CE,), jnp.int32),
            pltpu.VMEM((_N,), jnp.float32),
            pltpu.VMEM((_N,), jnp.float32),
            pltpu.VMEM((_N,), jnp.float32),
            pltpu.VMEM((_N,), jnp.float32),
            pltpu.VMEM((_N,), jnp.float32),
            pltpu.SemaphoreType.DMA,
            pltpu.SemaphoreType.DMA,
            pltpu.SemaphoreType.DMA,
            pltpu.SemaphoreType.DMA,
        ],
    )


_BLK = 10000


def _mlp_body(sums_r, cnt_r, sites_r, states_r,
              w1_r, b1_r, w2_r, b2_r, w3_r, b3_r, out_r):
    b = pl.program_id(0)
    cnt = jnp.sum(cnt_r[...], axis=0, keepdims=True)
    inv = 1.0 / jnp.maximum(cnt, 1.0)
    pool_t = sums_r[0] * inv

    sel = (lax.broadcasted_iota(jnp.int32, (1, _B), 1) == b)
    st_all = jnp.dot(states_r[...], w1_r[_D + _SL:, :],
                     preferred_element_type=jnp.float32)
    x_state = jnp.dot(sel.astype(jnp.float32), st_all,
                      preferred_element_type=jnp.float32)

    x_pool = lax.dot_general(pool_t, w1_r[0:_D, :],
                             (((0,), (0,)), ((), ())),
                             preferred_element_type=jnp.float32)
    h = (x_pool
         + jnp.dot(sites_r[0], w1_r[_D:_D + _SL, :],
                   preferred_element_type=jnp.float32)
         + x_state + b1_r[...])
    h = jnp.maximum(h, 0.0)
    h = jnp.maximum(jnp.dot(h, w2_r[...], preferred_element_type=jnp.float32)
                    + b2_r[...], 0.0)
    h = jnp.maximum(jnp.dot(h, w3_r[...], preferred_element_type=jnp.float32)
                    + b3_r[...], 0.0)
    out_r[0] = h


def _mlp_call(sums_t, cnt, sites, states, w1, b1, w2, b2, w3, b3):
    d_in = _D + _SL + _STL
    grid = (_B, pl.cdiv(_N, _BLK))
    return pl.pallas_call(
        _mlp_body,
        grid=grid,
        in_specs=[
            pl.BlockSpec((1, _D, _BLK), lambda b, n: (b, 0, n)),
            pl.BlockSpec((_NW, _BLK), lambda b, n: (0, n)),
            pl.BlockSpec((1, _BLK, _SL), lambda b, n: (b, n, 0)),
            pl.BlockSpec((_B, _STL), lambda b, n: (0, 0)),
            pl.BlockSpec((d_in, 128), lambda b, n: (0, 0)),
            pl.BlockSpec((1, 128), lambda b, n: (0, 0)),
            pl.BlockSpec((128, 128), lambda b, n: (0, 0)),
            pl.BlockSpec((1, 128), lambda b, n: (0, 0)),
            pl.BlockSpec((128, _SL), lambda b, n: (0, 0)),
            pl.BlockSpec((1, _SL), lambda b, n: (0, 0)),
        ],
        out_specs=pl.BlockSpec((1, _BLK, _SL), lambda b, n: (b, n, 0)),
        out_shape=jax.ShapeDtypeStruct((_B, _N, _SL), jnp.float32),
    )(sums_t, cnt, sites, states, w1, b1, w2, b2, w3, b3)


def kernel(sites, bonds, states, indices1, W1, b1, W2, b2, W3, b3):
    bonds_t = jnp.transpose(bonds, (0, 2, 1))
    sums_t, cnt = _sc_scatter()(bonds_t, indices1)
    return _mlp_call(sums_t, cnt, sites, states,
                     W1, b1.reshape(1, -1), W2, b2.reshape(1, -1),
                     W3, b3.reshape(1, -1))
